# Initial kernel scaffold; baseline (speedup 1.0000x reference)
import jax
import jax.numpy as jnp
from jax.experimental import pallas as pl


def kernel(x, y):
    raise NotImplementedError("write your pallas kernel here")


# speedup vs baseline: 45.3180x; 45.3180x over previous
import functools

import jax
import jax.numpy as jnp
from jax import lax
from jax.experimental import pallas as pl
from jax.experimental.pallas import tpu as pltpu
from jax.experimental.pallas import tpu_sc as plsc

B = 128
N = 100000
K = 5
L = 16
NC = 2
NS = 16
NW = NC * NS
RPW = B // NW
CH = 10000
NCHUNK = N // CH
UNROLL = 5
STEPS = CH // (L * UNROLL)
NEG = -3.0e38

_mesh = plsc.VectorSubcoreMesh(core_axis_name="c", subcore_axis_name="s")


@functools.partial(
    pl.kernel,
    mesh=_mesh,
    out_type=jax.ShapeDtypeStruct((NW * L,), jnp.float32),
    scratch_types=[
        pltpu.VMEM((CH,), jnp.float32),
        pltpu.VMEM((CH,), jnp.float32),
        pltpu.VMEM((2 * L,), jnp.int32),
        pltpu.VMEM((2 * L,), jnp.float32),
        pltpu.VMEM((L,), jnp.float32),
        pltpu.SemaphoreType.DMA,
        pltpu.SemaphoreType.DMA,
    ],
)
def _loss_sc(x_hbm, y_hbm, out_hbm, buf0, buf1, ybuf, sbuf, obuf, sem0, sem1):
    bufs = (buf0, buf1)
    wid = lax.axis_index("s") * NC + lax.axis_index("c")
    lane_ids = lax.iota(jnp.int32, L)

    gdims = lax.GatherDimensionNumbers(
        offset_dims=(), collapsed_slice_dims=(0,), start_index_map=(0,))

    def shuf(v, s):
        return lax.gather(
            v, (lane_ids ^ s)[:, None], gdims, (1,),
            mode=lax.GatherScatterMode.PROMISE_IN_BOUNDS)

    def bmax(v):
        for s in (1, 2, 4, 8):
            v = jnp.maximum(v, shuf(v, s))
        return v

    def bmin(v):
        for s in (1, 2, 4, 8):
            v = jnp.minimum(v, shuf(v, s))
        return v

    pltpu.sync_copy(y_hbm.at[pl.ds(pl.multiple_of((wid // 4) * L, 8), L)],
                    ybuf.at[pl.ds(0, L)])
    sems = (sem0, sem1)

    def chunk_body(pb, i, carry):
        a = list(carry)
        base = i * (L * UNROLL)
        for u in range(UNROLL):
            t = bufs[pb][pl.ds(base + u * L, L)]
            for k in range(6):
                hi = jnp.maximum(a[k], t)
                t = jnp.minimum(a[k], t)
                a[k] = hi
        return tuple(a)

    def row_body(r, acc):
        row = wid * RPW + r

        lane = (wid % 4) * RPW + r
        y_val = ybuf[pl.ds(lane, L)][0]
        a_off = pl.multiple_of(
            row * N + jnp.minimum((y_val >> 3) << 3, N - L), 8)
        pltpu.sync_copy(x_hbm.at[pl.ds(a_off, L)], sbuf.at[pl.ds(0, L)])
        s_lane = y_val - (a_off - row * N)
        s_v = jnp.broadcast_to(sbuf[pl.ds(s_lane, L)][0], (L,))

        carry = tuple(jnp.full((L,), NEG) for _ in range(6))
        pend = [None, None]
        pend[0] = pltpu.async_copy(
            x_hbm.at[pl.ds(pl.multiple_of(row * N, 8), CH)], buf0, sems[0])
        for c in range(NCHUNK):
            pb = c % 2
            if c + 1 < NCHUNK:
                nb = (c + 1) % 2
                pend[nb] = pltpu.async_copy(
                    x_hbm.at[pl.ds(pl.multiple_of(
                        row * N + (c + 1) * CH, 8), CH)], bufs[nb],
                    sems[nb])
            pend[pb].wait()
            carry = lax.fori_loop(
                0, STEPS, functools.partial(chunk_body, pb), carry)

        a = list(carry)
        csum_v = jnp.zeros((L,), jnp.float32)
        c_v = csum_v
        m_v = csum_v
        for j in range(6):
            m_v = bmax(a[0])
            c_v = jnp.maximum(m_v + 1.0 - s_v, 0.0)
            csum_v = csum_v + c_v
            if j < 5:
                eq = a[0] == m_v
                pm = lane_ids == bmin(jnp.where(eq, lane_ids, L))
                for k in range(5):
                    a[k] = jnp.where(pm, a[k + 1], a[k])
                a[5] = jnp.where(pm, jnp.full((L,), NEG), a[5])
        sub_v = jnp.where(s_v >= m_v, jnp.full((L,), jnp.float32(1.0)), c_v)
        return acc + (csum_v - sub_v)

    acc_v = lax.fori_loop(0, RPW, row_body, jnp.zeros((L,), jnp.float32))
    obuf[...] = acc_v * jnp.float32(1.0 / (K * B))
    pltpu.sync_copy(obuf, out_hbm.at[pl.ds(pl.multiple_of(wid * L, 8), L)])


def kernel(x, y):
    parts = _loss_sc(x.reshape(-1), y.astype(jnp.int32))
    return jnp.sum(parts.reshape(NW, L)[:, 0])


# speedup vs baseline: 66.8264x; 1.4746x over previous
import functools

import jax
import jax.numpy as jnp
from jax import lax
from jax.experimental import pallas as pl
from jax.experimental.pallas import tpu as pltpu
from jax.experimental.pallas import tpu_sc as plsc

B = 128
N = 100000
NPAD = 100096
K = 5
L = 16
NC = 2
NS = 16
NW = NC * NS
CH = 5888
NCHUNK = NPAD // CH
U = 4
FULL_IT = CH // (L * U)
TAIL_VEC = (N - (NCHUNK - 1) * CH) // L
NEG = -3.0e38

_mesh = plsc.VectorSubcoreMesh(core_axis_name="c", subcore_axis_name="s")


@functools.partial(
    pl.kernel,
    mesh=_mesh,
    out_type=jax.ShapeDtypeStruct((NW * L,), jnp.float32),
    scratch_types=[
        pltpu.VMEM((8, CH), jnp.float32),
        pltpu.VMEM((8, CH), jnp.float32),
        pltpu.VMEM((8, 128), jnp.float32),
        pltpu.VMEM((2 * L,), jnp.int32),
        pltpu.VMEM((2 * L,), jnp.float32),
        pltpu.VMEM((L,), jnp.float32),
        pltpu.SemaphoreType.DMA,
        pltpu.SemaphoreType.DMA,
    ],
)
def _loss_sc(x_hbm, y_hbm, out_hbm, buf0, buf1, tbuf, ybuf, xbuf, obuf,
             sem0, sem1):
    bufs = (buf0, buf1)
    sems = (sem0, sem1)
    cid = lax.axis_index("c")
    sid = lax.axis_index("s")
    wid = sid * NC + cid
    rg = cid * 8 + (sid >> 1)
    h = sid & 1
    lane_ids = lax.iota(jnp.int32, L)

    gdims = lax.GatherDimensionNumbers(
        offset_dims=(), collapsed_slice_dims=(0,), start_index_map=(0,))

    def shuf(v, s):
        return lax.gather(
            v, (lane_ids ^ s)[:, None], gdims, (1,),
            mode=lax.GatherScatterMode.PROMISE_IN_BOUNDS)

    def bmax(v):
        for s in (1, 2, 4, 8):
            v = jnp.maximum(v, shuf(v, s))
        return v

    def bmin(v):
        for s in (1, 2, 4, 8):
            v = jnp.minimum(v, shuf(v, s))
        return v

    ybase = pl.multiple_of((rg >> 1) * L, 16)
    pltpu.sync_copy(y_hbm.at[pl.ds(ybase, L)], ybuf.at[pl.ds(0, L)])

    def insert(a, t):
        for k in range(6):
            hi = jnp.maximum(a[k], t)
            t = jnp.minimum(a[k], t)
            a[k] = hi

    def chunk_body(pb, rib, i, carry):
        a = list(carry)
        base = i * (L * U)
        for u in range(U):
            insert(a, bufs[pb][rib, pl.ds(base + u * L, L)])
        return tuple(a)

    row_base = pl.multiple_of(rg * 8, 8)
    accs = [[jnp.full((L,), NEG, jnp.float32)] * 6 for _ in range(4)]

    pend = [None, None]
    pend[0] = pltpu.async_copy(
        x_hbm.at[pl.ds(row_base, 8), pl.ds(0, CH)], buf0, sems[0])
    for ci in range(NCHUNK):
        pb = ci % 2
        if ci + 1 < NCHUNK:
            nb = (ci + 1) % 2
            pend[nb] = pltpu.async_copy(
                x_hbm.at[pl.ds(row_base, 8),
                         pl.ds(pl.multiple_of((ci + 1) * CH, 128), CH)],
                bufs[nb], sems[nb])
        pend[pb].wait()
        n_it = FULL_IT if ci + 1 < NCHUNK else TAIL_VEC // U
        for r in range(4):
            rib = h * 4 + r
            accs[r] = list(lax.fori_loop(
                0, n_it, functools.partial(chunk_body, pb, rib),
                tuple(accs[r])))
            if ci + 1 == NCHUNK:
                for v in range(n_it * U, TAIL_VEC):
                    insert(accs[r], bufs[pb][rib, pl.ds(v * L, L)])

    acc_v = jnp.zeros((L,), jnp.float32)
    for r in range(4):
        rib = h * 4 + r
        lane = ((rg & 1) * 8 + rib).astype(jnp.int32)
        y_val = ybuf[pl.ds(lane, L)][0]
        pltpu.sync_copy(
            x_hbm.at[pl.ds(row_base, 8),
                     pl.ds(pl.multiple_of((y_val >> 7) << 7, 128), 128)],
            tbuf)
        ylo = y_val & 127
        st = (ylo >> 4) << 4
        xbuf[pl.ds(0, L)] = tbuf[rib, pl.ds(st, L)]
        s_v = jnp.broadcast_to(xbuf[pl.ds(ylo & 15, L)][0], (L,))

        a = accs[r]
        csum_v = jnp.zeros((L,), jnp.float32)
        c_v = csum_v
        m_v = csum_v
        for j in range(6):
            m_v = bmax(a[0])
            c_v = jnp.maximum(m_v + 1.0 - s_v, 0.0)
            csum_v = csum_v + c_v
            if j < 5:
                eq = a[0] == m_v
                pm = lane_ids == bmin(jnp.where(eq, lane_ids, L))
                for k in range(5):
                    a[k] = jnp.where(pm, a[k + 1], a[k])
                a[5] = jnp.where(pm, jnp.full((L,), NEG, jnp.float32), a[5])
        sub_v = jnp.where(s_v >= m_v, jnp.full((L,), jnp.float32(1.0)), c_v)
        acc_v = acc_v + (csum_v - sub_v)

    obuf[...] = acc_v * jnp.float32(1.0 / (K * B))
    pltpu.sync_copy(obuf, out_hbm.at[pl.ds(pl.multiple_of(wid * L, 8), L)])


def kernel(x, y):
    parts = _loss_sc(x, y.astype(jnp.int32))
    return jnp.sum(parts.reshape(NW, L)[:, 0])
